# baseline (speedup 1.0000x reference)
import jax
import jax.numpy as jnp
from jax.experimental import pallas as pl
from jax.experimental.pallas import tpu as pltpu


def _fused_kernel(x_ref,
                  xn_ref,
                  wih0f_ref,
                  b0_ref,
                  whh0f_ref,
                  wih1f_ref,
                  whh1f_ref,
                  b1_ref,
                  wfc_ref,
                  bfc_ref,
                  out_ref,
                  ga_ref,
                  gb_ref,
                  wih0_ref,
                  whh0_ref,
                  w1_ref,
                  g1p_ref,
                  c0_ref, c1_ref,
                  m0_ref):
    chunk = pl.program_id(0)
    nchunks = pl.num_programs(0)
    B, H = c0_ref.shape
    four_h = 4 * H
    tc = ga_ref.shape[0]
    F = x_ref.shape[2]

    def project_group(src_ref, dst_ref, lo, hi):
        bg = hi - lo
        xm = src_ref[lo:hi, :, :].reshape(bg * tc, F).astype(jnp.bfloat16)
        gp = jnp.dot(xm, wih0_ref[...], preferred_element_type=jnp.float32)
        gp = gp + b0_ref[...]
        dst_ref[:, lo:hi, :] = jnp.swapaxes(
            gp.reshape(bg, tc, four_h).astype(jnp.bfloat16), 0, 1)

    @pl.when(chunk == 0)
    def _():
        wih0_ref[...] = wih0f_ref[...].astype(jnp.bfloat16)
        whh0_ref[...] = whh0f_ref[...].astype(jnp.bfloat16)
        w1_ref[0 * H:1 * H] = wih1f_ref[...].astype(jnp.bfloat16)
        w1_ref[1 * H:2 * H] = whh1f_ref[...].astype(jnp.bfloat16)
        g1p_ref[...] = jnp.zeros_like(g1p_ref)
        c0_ref[...] = jnp.zeros_like(c0_ref)
        c1_ref[...] = jnp.zeros_like(c1_ref)
        m0_ref[...] = jnp.zeros_like(m0_ref)
        project_group(x_ref, ga_ref, 0, B)

    b1 = jnp.broadcast_to(b1_ref[...], (B, four_h))

    def sig(v):
        return 0.5 * jnp.tanh(0.5 * v) + 0.5

    def act(g):
        i = sig(g[:, 0 * H:1 * H])
        f = sig(g[:, 1 * H:2 * H])
        gg = jnp.tanh(g[:, 2 * H:3 * H])
        o = sig(g[:, 3 * H:4 * H])
        return i, f, gg, o

    def l1_finish(g1p, c1):
        i1, f1, g1g, o1 = act(g1p)
        c1n = f1 * c1 + i1 * g1g
        h1 = (o1 * jnp.tanh(c1n)).astype(jnp.bfloat16)
        return h1, c1n

    def run(read_ref, write_ref):
        def step(t, carry):
            g1p, c0, c1, m0 = carry

            h1, c1n = l1_finish(g1p, c1)

            g0 = read_ref[t].astype(jnp.float32) + m0
            i0, f0, g0g, o0 = act(g0)
            c0n = f0 * c0 + i0 * g0g
            h0n = (o0 * jnp.tanh(c0n)).astype(jnp.bfloat16)

            m0n = jnp.dot(h0n, whh0_ref[...],
                          preferred_element_type=jnp.float32)
            z = jnp.concatenate([h0n, h1], axis=1)
            g1n = (jnp.dot(z, w1_ref[...],
                           preferred_element_type=jnp.float32) + b1)

            return g1n, c0n, c1n, m0n

        bg = B // 4
        stages = {(tc * (2 * k + 1)) // 8: k for k in range(4)}

        carry = (g1p_ref[...], c0_ref[...], c1_ref[...], m0_ref[...])
        for t in range(tc):
            carry = step(t, carry)
            if t in stages:
                k = stages[t]
                project_group(xn_ref, write_ref, k * bg, (k + 1) * bg)
        g1p, c0n, c1n, m0n = carry

        g1p_ref[...] = g1p
        c0_ref[...] = c0n
        c1_ref[...] = c1n
        m0_ref[...] = m0n

        @pl.when(chunk == nchunks - 1)
        def _():
            h1_fin, _ = l1_finish(g1p, c1n)
            out_ref[...] = (jnp.dot(h1_fin.astype(jnp.float32), wfc_ref[...],
                                    preferred_element_type=jnp.float32)
                            + bfc_ref[...])

    @pl.when(chunk % 2 == 0)
    def _():
        run(ga_ref, gb_ref)

    @pl.when(chunk % 2 == 1)
    def _():
        run(gb_ref, ga_ref)


def _pick_chunk(T, target):
    best = None
    for tc in range(1, T + 1):
        if T % tc == 0 and tc <= target and (tc % 8 == 0 or best is None):
            best = tc
    return best if best is not None else T


def kernel(x, wih0, whh0, b0, wih1, whh1, b1, wfc, bfc):
    B, T, F = x.shape
    H = whh0.shape[0]
    four_h = 4 * H
    O = wfc.shape[1]

    tc = _pick_chunk(T, 40)
    nc = T // tc

    out = pl.pallas_call(
        _fused_kernel,
        out_shape=jax.ShapeDtypeStruct((B, O), jnp.float32),
        grid=(nc,),
        in_specs=[
            pl.BlockSpec((B, tc, F), lambda c: (0, c, 0)),
            pl.BlockSpec((B, tc, F),
                         lambda c: (0, jnp.minimum(c + 1, nc - 1), 0)),
            pl.BlockSpec((F, four_h), lambda c: (0, 0)),
            pl.BlockSpec((1, four_h), lambda c: (0, 0)),
            pl.BlockSpec((H, four_h), lambda c: (0, 0)),
            pl.BlockSpec((H, four_h), lambda c: (0, 0)),
            pl.BlockSpec((H, four_h), lambda c: (0, 0)),
            pl.BlockSpec((1, four_h), lambda c: (0, 0)),
            pl.BlockSpec((H, O), lambda c: (0, 0)),
            pl.BlockSpec((1, O), lambda c: (0, 0)),
        ],
        out_specs=pl.BlockSpec((B, O), lambda c: (0, 0)),
        scratch_shapes=[
            pltpu.VMEM((tc, B, four_h), jnp.bfloat16),
            pltpu.VMEM((tc, B, four_h), jnp.bfloat16),
            pltpu.VMEM((F, four_h), jnp.bfloat16),
            pltpu.VMEM((H, four_h), jnp.bfloat16),
            pltpu.VMEM((2 * H, four_h), jnp.bfloat16),
            pltpu.VMEM((B, four_h), jnp.float32),
            pltpu.VMEM((B, H), jnp.float32),
            pltpu.VMEM((B, H), jnp.float32),
            pltpu.VMEM((B, four_h), jnp.float32),
        ],
        compiler_params=pltpu.CompilerParams(
            dimension_semantics=("arbitrary",)),
    )(x, x, wih0, b0, whh0, wih1, whh1, b1, wfc, bfc)

    return out[:, None, :]


# speedup vs baseline: 1.0521x; 1.0521x over previous
import jax
import jax.numpy as jnp
from jax.experimental import pallas as pl
from jax.experimental.pallas import tpu as pltpu


def _fused_kernel(x_ref,
                  wih0f_ref,
                  b0_ref,
                  whh0f_ref,
                  wih1f_ref,
                  whh1f_ref,
                  b1_ref,
                  wfc_ref,
                  bfc_ref,
                  out_ref,
                  gih_ref,
                  wih0_ref,
                  whh0_ref,
                  w1_ref,
                  g1p_ref,
                  c0_ref, c1_ref,
                  m0_ref):
    chunk = pl.program_id(0)
    B, H = c0_ref.shape
    four_h = 4 * H
    tc = gih_ref.shape[0]

    @pl.when(chunk == 0)
    def _():
        wih0_ref[...] = wih0f_ref[...].astype(jnp.bfloat16)
        whh0_ref[...] = whh0f_ref[...].astype(jnp.bfloat16)
        w1_ref[0 * H:1 * H] = wih1f_ref[...].astype(jnp.bfloat16)
        w1_ref[1 * H:2 * H] = whh1f_ref[...].astype(jnp.bfloat16)
        g1p_ref[...] = jnp.zeros_like(g1p_ref)
        c0_ref[...] = jnp.zeros_like(c0_ref)
        c1_ref[...] = jnp.zeros_like(c1_ref)
        m0_ref[...] = jnp.zeros_like(m0_ref)

    xm = x_ref[...].reshape(B * tc, x_ref.shape[2]).astype(jnp.bfloat16)
    gp = jnp.dot(xm, wih0_ref[...], preferred_element_type=jnp.float32)
    gp = gp + b0_ref[...]
    gih_ref[...] = jnp.swapaxes(
        gp.reshape(B, tc, four_h).astype(jnp.bfloat16), 0, 1)

    b1 = jnp.broadcast_to(b1_ref[...], (B, four_h))

    def sig(v):
        return 0.5 * jnp.tanh(0.5 * v) + 0.5

    def act(g):
        i = sig(g[:, 0 * H:1 * H])
        f = sig(g[:, 1 * H:2 * H])
        gg = jnp.tanh(g[:, 2 * H:3 * H])
        o = sig(g[:, 3 * H:4 * H])
        return i, f, gg, o

    def l1_finish(g1p, c1):
        i1, f1, g1g, o1 = act(g1p)
        c1n = f1 * c1 + i1 * g1g
        h1 = (o1 * jnp.tanh(c1n)).astype(jnp.bfloat16)
        return h1, c1n

    def step(t, carry):
        g1p, c0, c1, m0 = carry

        h1, c1n = l1_finish(g1p, c1)

        g0 = gih_ref[t].astype(jnp.float32) + m0
        i0, f0, g0g, o0 = act(g0)
        c0n = f0 * c0 + i0 * g0g
        h0n = (o0 * jnp.tanh(c0n)).astype(jnp.bfloat16)

        m0n = jnp.dot(h0n, whh0_ref[...], preferred_element_type=jnp.float32)
        z = jnp.concatenate([h0n, h1], axis=1)
        g1n = jnp.dot(z, w1_ref[...], preferred_element_type=jnp.float32) + b1

        return g1n, c0n, c1n, m0n

    carry = (g1p_ref[...], c0_ref[...], c1_ref[...], m0_ref[...])
    for t in range(tc):
        carry = step(t, carry)
    g1p, c0n, c1n, m0n = carry

    g1p_ref[...] = g1p
    c0_ref[...] = c0n
    c1_ref[...] = c1n
    m0_ref[...] = m0n

    @pl.when(chunk == pl.num_programs(0) - 1)
    def _():
        h1_fin, _ = l1_finish(g1p, c1n)
        out_ref[...] = (jnp.dot(h1_fin.astype(jnp.float32), wfc_ref[...],
                                preferred_element_type=jnp.float32)
                        + bfc_ref[...])


def _pick_chunk(T, target):
    best = None
    for tc in range(1, T + 1):
        if T % tc == 0 and tc <= target and (tc % 8 == 0 or best is None):
            best = tc
    return best if best is not None else T


def kernel(x, wih0, whh0, b0, wih1, whh1, b1, wfc, bfc):
    B, T, F = x.shape
    H = whh0.shape[0]
    four_h = 4 * H
    O = wfc.shape[1]

    tc = _pick_chunk(T, 40)
    nc = T // tc

    out = pl.pallas_call(
        _fused_kernel,
        out_shape=jax.ShapeDtypeStruct((B, O), jnp.float32),
        grid=(nc,),
        in_specs=[
            pl.BlockSpec((B, tc, F), lambda c: (0, c, 0)),
            pl.BlockSpec((F, four_h), lambda c: (0, 0)),
            pl.BlockSpec((1, four_h), lambda c: (0, 0)),
            pl.BlockSpec((H, four_h), lambda c: (0, 0)),
            pl.BlockSpec((H, four_h), lambda c: (0, 0)),
            pl.BlockSpec((H, four_h), lambda c: (0, 0)),
            pl.BlockSpec((1, four_h), lambda c: (0, 0)),
            pl.BlockSpec((H, O), lambda c: (0, 0)),
            pl.BlockSpec((1, O), lambda c: (0, 0)),
        ],
        out_specs=pl.BlockSpec((B, O), lambda c: (0, 0)),
        scratch_shapes=[
            pltpu.VMEM((tc, B, four_h), jnp.bfloat16),
            pltpu.VMEM((F, four_h), jnp.bfloat16),
            pltpu.VMEM((H, four_h), jnp.bfloat16),
            pltpu.VMEM((2 * H, four_h), jnp.bfloat16),
            pltpu.VMEM((B, four_h), jnp.float32),
            pltpu.VMEM((B, H), jnp.float32),
            pltpu.VMEM((B, H), jnp.float32),
            pltpu.VMEM((B, four_h), jnp.float32),
        ],
        compiler_params=pltpu.CompilerParams(
            dimension_semantics=("arbitrary",)),
    )(x, wih0, b0, whh0, wih1, whh1, b1, wfc, bfc)

    return out[:, None, :]
